# baseline (speedup 1.0000x reference)
import functools

import jax
import jax.numpy as jnp
from jax import lax
from jax.experimental import pallas as pl
from jax.experimental.pallas import tpu as pltpu
from jax.experimental.pallas import tpu_sc as plsc

N = 10000
D = 128
E = 320000
NC = 2
NS = 16
NW = NC * NS
B = 128
STEPS = 80
CHUNKS = 2
CSTEPS = STEPS // CHUNKS
EPAD = NW * STEPS * B
NDUMP = 240
NPAD = N + NDUMP
RPT = NPAD // NS
R = 2000


@functools.lru_cache(maxsize=1)
def _build_sc_kernels():
    mesh = plsc.VectorSubcoreMesh(core_axis_name="c", subcore_axis_name="s",
                                  num_cores=NC, num_subcores=NS)

    @functools.partial(
        pl.kernel,
        out_type=jax.ShapeDtypeStruct((NC, NPAD, D), jnp.float32),
        mesh=mesh,
        scratch_types=[
            pltpu.VMEM((CSTEPS, B), jnp.int32),
            pltpu.VMEM((CSTEPS, B), jnp.int32),
            pltpu.VMEM((B, D), jnp.float32),
            pltpu.VMEM((B, D), jnp.float32),
            pltpu.VMEM_SHARED((NPAD, D), jnp.float32),
            pltpu.SemaphoreType.DMA,
            pltpu.SemaphoreType.DMA,
        ],
    )
    def sc_conv(hs_hbm, src_hbm, dst_hbm, out_hbm,
                src_v, dst_v, buf0, buf1, acc_sh, sem0, sem1):
        c = lax.axis_index("c")
        s = lax.axis_index("s")
        zv = jnp.zeros((16,), jnp.float32)

        def fill_zero(j, carry):
            for g in range(8):
                buf0[j, pl.ds(16 * g, 16)] = zv
            return carry

        lax.fori_loop(0, B, fill_zero, 0)
        for q in range(RPT // B):
            pltpu.sync_copy(buf0, acc_sh.at[pl.ds(s * RPT + q * B, B)])
        bufs = (buf0, buf1)
        sems = (sem0, sem1)
        plsc.subcore_barrier()

        wid = s * NC + c
        for chunk in range(CHUNKS):
            pltpu.sync_copy(src_hbm.at[wid, pl.ds(chunk * CSTEPS, CSTEPS)],
                            src_v)
            pltpu.sync_copy(dst_hbm.at[wid, pl.ds(chunk * CSTEPS, CSTEPS)],
                            dst_v)
            pltpu.async_copy(hs_hbm.at[src_v.at[0]], buf0, sem0)
            pltpu.async_copy(hs_hbm.at[src_v.at[1]], buf1, sem1)

            def step(g, carry):
                for b in range(2):
                    j = g * 2 + b
                    pltpu.make_async_copy(hs_hbm.at[src_v.at[j]], bufs[b],
                                          sems[b]).wait()
                    pltpu.sync_copy(bufs[b], acc_sh.at[dst_v.at[j]], add=True)
                    nxt = j + 2

                    @pl.when(nxt < CSTEPS)
                    def _():
                        pltpu.async_copy(hs_hbm.at[src_v.at[nxt]], bufs[b],
                                         sems[b])
                return carry

            lax.fori_loop(0, CSTEPS // 2, step, 0)
        plsc.subcore_barrier()
        pltpu.sync_copy(acc_sh.at[pl.ds(s * RPT, RPT)],
                        out_hbm.at[c, pl.ds(s * RPT, RPT)])

    return sc_conv


def _tc1_body(x_ref, w1_ref, degp_ref, hs1_ref, dinv_ref):
    deg = degp_ref[0] + degp_ref[1] + 1.0
    dinv = lax.rsqrt(deg)
    h = jnp.dot(x_ref[...], w1_ref[...], preferred_element_type=jnp.float32)
    hs1_ref[...] = h * dinv
    dinv_ref[...] = dinv


def _tc2_body(accp_ref, hs_ref, dinv_ref, b_ref, lnw_ref, lnb_ref, w2_ref,
              out_ref):
    dinv = dinv_ref[...]
    acc = accp_ref[0] + accp_ref[1] + hs_ref[...]
    out = acc * dinv + b_ref[...]
    mu = jnp.mean(out, axis=-1, keepdims=True)
    var = jnp.mean((out - mu) ** 2, axis=-1, keepdims=True)
    h = (out - mu) * lax.rsqrt(var + 1e-5) * lnw_ref[...] + lnb_ref[...]
    h = jnp.maximum(h, 0.0)
    h2 = jnp.dot(h, w2_ref[...], preferred_element_type=jnp.float32)
    out_ref[...] = h2 * dinv


def _tc3_body(accp_ref, hs_ref, dinv_ref, b_ref, lnw_ref, lnb_ref,
              fc1w_ref, fc1b_ref, fc2w_ref, fc2b_ref, out_ref):
    acc = accp_ref[0] + accp_ref[1] + hs_ref[...]
    out = acc * dinv_ref[...] + b_ref[...]
    mu = jnp.mean(out, axis=-1, keepdims=True)
    var = jnp.mean((out - mu) ** 2, axis=-1, keepdims=True)
    h = (out - mu) * lax.rsqrt(var + 1e-5) * lnw_ref[...] + lnb_ref[...]
    h = jnp.maximum(h, 0.0)
    t = jnp.dot(h, fc1w_ref[...], preferred_element_type=jnp.float32)
    t = jnp.maximum(t + fc1b_ref[...], 0.0)
    logits = jnp.dot(t, fc2w_ref[...], preferred_element_type=jnp.float32)
    logits = logits + fc2b_ref[...]
    m = jnp.max(logits, axis=-1, keepdims=True)
    lse = m + jnp.log(jnp.sum(jnp.exp(logits - m), axis=-1, keepdims=True))
    out_ref[...] = logits - lse


def _row_spec(shape):
    return pl.BlockSpec(shape, lambda i: (i, 0))


def _full_spec(shape):
    return pl.BlockSpec(shape, lambda i: tuple(0 for _ in shape))


def _part_spec(last):
    return pl.BlockSpec((NC, R, last), lambda i: (0, i, 0))


_GRID = (N // R,)

_tc1 = pl.pallas_call(
    _tc1_body,
    grid=_GRID,
    in_specs=[_row_spec((R, D)), _full_spec((D, D)), _part_spec(D)],
    out_specs=[_row_spec((R, D)), _row_spec((R, D))],
    out_shape=[jax.ShapeDtypeStruct((N, D), jnp.float32),
               jax.ShapeDtypeStruct((N, D), jnp.float32)],
)

_tc2 = pl.pallas_call(
    _tc2_body,
    grid=_GRID,
    in_specs=[_part_spec(D), _row_spec((R, D)), _row_spec((R, D)),
              _full_spec((1, D)), _full_spec((1, D)), _full_spec((1, D)),
              _full_spec((D, D))],
    out_specs=_row_spec((R, D)),
    out_shape=jax.ShapeDtypeStruct((N, D), jnp.float32),
)

_tc3 = pl.pallas_call(
    _tc3_body,
    grid=_GRID,
    in_specs=[_part_spec(D), _row_spec((R, D)), _row_spec((R, D)),
              _full_spec((1, D)), _full_spec((1, D)), _full_spec((1, D)),
              _full_spec((D, D // 2)), _full_spec((1, D // 2)),
              _full_spec((D // 2, 40)), _full_spec((1, 40))],
    out_specs=_row_spec((R, 40)),
    out_shape=jax.ShapeDtypeStruct((N, 40), jnp.float32),
)


def kernel(x, edge_index, W1, b1, ln1_w, ln1_b, W2, b2, ln2_w, ln2_b,
           fc1_w, fc1_b, fc2_w, fc2_b):
    src = edge_index[0].astype(jnp.int32)
    dst = edge_index[1].astype(jnp.int32)
    npad = EPAD - E
    src_p = jnp.concatenate([src, jnp.zeros((npad,), jnp.int32)])
    dst_p = jnp.concatenate(
        [dst, N + (jnp.arange(npad, dtype=jnp.int32) % NDUMP)])
    src_idx = src_p.reshape(NW, STEPS, B)
    dst_idx = dst_p.reshape(NW, STEPS, B)

    _sc_conv = _build_sc_kernels()
    degp = _sc_conv(jnp.ones((N, D), jnp.float32),
                    jnp.zeros_like(src_idx), dst_idx)
    hs1, dinv = _tc1(x, W1, degp)
    acc1 = _sc_conv(hs1, src_idx, dst_idx)
    hs2 = _tc2(acc1, hs1, dinv, b1.reshape(1, D), ln1_w.reshape(1, D),
               ln1_b.reshape(1, D), W2)
    acc2 = _sc_conv(hs2, src_idx, dst_idx)
    out = _tc3(acc2, hs2, dinv, b2.reshape(1, D), ln2_w.reshape(1, D),
               ln2_b.reshape(1, D), fc1_w, fc1_b.reshape(1, D // 2),
               fc2_w, fc2_b.reshape(1, 40))
    return out


# speedup vs baseline: 25.7063x; 25.7063x over previous
import functools

import jax
import jax.numpy as jnp
from jax import lax
from jax.experimental import pallas as pl
from jax.experimental.pallas import tpu as pltpu
from jax.experimental.pallas import tpu_sc as plsc

N = 10000
D = 128
E = 320000
NC = 2
NS = 16
NW = NC * NS
B = 128
STEPS = 80
CHUNKS = 2
CSTEPS = STEPS // CHUNKS
EPAD = NW * STEPS * B
NDUMP = 240
NPAD = N + NDUMP
RPT = NPAD // NS
R = 2000


@functools.lru_cache(maxsize=1)
def _build_sc_kernels():
    mesh = plsc.VectorSubcoreMesh(core_axis_name="c", subcore_axis_name="s",
                                  num_cores=NC, num_subcores=NS)

    @functools.partial(
        pl.kernel,
        out_type=jax.ShapeDtypeStruct((NC, NPAD, D), jnp.float32),
        mesh=mesh,
        scratch_types=[
            pltpu.VMEM((CSTEPS, B), jnp.int32),
            pltpu.VMEM((CSTEPS, B), jnp.int32),
            pltpu.VMEM((B, D), jnp.float32),
            pltpu.VMEM((B, D), jnp.float32),
            pltpu.VMEM_SHARED((NPAD, D), jnp.float32),
            pltpu.SemaphoreType.DMA,
            pltpu.SemaphoreType.DMA,
        ],
    )
    def sc_conv(hs_hbm, src_hbm, dst_hbm, out_hbm,
                src_v, dst_v, buf0, buf1, acc_sh, sem0, sem1):
        c = lax.axis_index("c")
        s = lax.axis_index("s")
        zv = jnp.zeros((16,), jnp.float32)

        def fill_zero(j, carry):
            for g in range(8):
                buf0[j, pl.ds(16 * g, 16)] = zv
            return carry

        lax.fori_loop(0, B, fill_zero, 0)
        for q in range(RPT // B):
            pltpu.sync_copy(buf0, acc_sh.at[pl.ds(s * RPT + q * B, B)])
        bufs = (buf0, buf1)
        sems = (sem0, sem1)
        plsc.subcore_barrier()

        wid = s * NC + c
        for chunk in range(CHUNKS):
            pltpu.sync_copy(src_hbm.at[wid, pl.ds(chunk * CSTEPS, CSTEPS)],
                            src_v)
            pltpu.sync_copy(dst_hbm.at[wid, pl.ds(chunk * CSTEPS, CSTEPS)],
                            dst_v)
            pltpu.async_copy(hs_hbm.at[src_v.at[0]], buf0, sem0)
            pltpu.async_copy(hs_hbm.at[src_v.at[1]], buf1, sem1)

            def step(g, carry):
                for b in range(2):
                    j = g * 2 + b
                    pltpu.make_async_copy(hs_hbm.at[src_v.at[j]], bufs[b],
                                          sems[b]).wait()
                    pltpu.sync_copy(bufs[b], acc_sh.at[dst_v.at[j]], add=True)
                    nxt = j + 2

                    @pl.when(nxt < CSTEPS)
                    def _():
                        pltpu.async_copy(hs_hbm.at[src_v.at[nxt]], bufs[b],
                                         sems[b])
                return carry

            lax.fori_loop(0, CSTEPS // 2, step, 0)
        plsc.subcore_barrier()
        pltpu.sync_copy(acc_sh.at[pl.ds(s * RPT, RPT)],
                        out_hbm.at[c, pl.ds(s * RPT, RPT)])

    return sc_conv


def _tc1_body(x_ref, w1_ref, degp_ref, hs1_ref, dinv_ref):
    deg = degp_ref[0] + degp_ref[1] + 1.0
    dinv = lax.rsqrt(deg)
    h = jnp.dot(x_ref[...], w1_ref[...], preferred_element_type=jnp.float32)
    hs1_ref[...] = h * dinv
    dinv_ref[...] = dinv


def _tc2_body(accp_ref, hs_ref, dinv_ref, b_ref, lnw_ref, lnb_ref, w2_ref,
              out_ref):
    dinv = dinv_ref[...]
    acc = accp_ref[0] + accp_ref[1] + hs_ref[...]
    out = acc * dinv + b_ref[...]
    mu = jnp.mean(out, axis=-1, keepdims=True)
    var = jnp.mean((out - mu) ** 2, axis=-1, keepdims=True)
    h = (out - mu) * lax.rsqrt(var + 1e-5) * lnw_ref[...] + lnb_ref[...]
    h = jnp.maximum(h, 0.0)
    h2 = jnp.dot(h, w2_ref[...], preferred_element_type=jnp.float32)
    out_ref[...] = h2 * dinv


def _tc3_body(accp_ref, hs_ref, dinv_ref, b_ref, lnw_ref, lnb_ref,
              fc1w_ref, fc1b_ref, fc2w_ref, fc2b_ref, out_ref):
    acc = accp_ref[0] + accp_ref[1] + hs_ref[...]
    out = acc * dinv_ref[...] + b_ref[...]
    mu = jnp.mean(out, axis=-1, keepdims=True)
    var = jnp.mean((out - mu) ** 2, axis=-1, keepdims=True)
    h = (out - mu) * lax.rsqrt(var + 1e-5) * lnw_ref[...] + lnb_ref[...]
    h = jnp.maximum(h, 0.0)
    t = jnp.dot(h, fc1w_ref[...], preferred_element_type=jnp.float32)
    t = jnp.maximum(t + fc1b_ref[...], 0.0)
    logits = jnp.dot(t, fc2w_ref[...], preferred_element_type=jnp.float32)
    logits = logits + fc2b_ref[...]
    m = jnp.max(logits, axis=-1, keepdims=True)
    lse = m + jnp.log(jnp.sum(jnp.exp(logits - m), axis=-1, keepdims=True))
    out_ref[...] = logits - lse


def _row_spec(shape):
    return pl.BlockSpec(shape, lambda i: (i, 0))


def _full_spec(shape):
    return pl.BlockSpec(shape, lambda i: tuple(0 for _ in shape))


def _part_spec(last):
    return pl.BlockSpec((NC, R, last), lambda i: (0, i, 0))


_GRID = (N // R,)

_tc1 = pl.pallas_call(
    _tc1_body,
    grid=_GRID,
    in_specs=[_row_spec((R, D)), _full_spec((D, D)), _part_spec(D)],
    out_specs=[_row_spec((R, D)), _row_spec((R, D))],
    out_shape=[jax.ShapeDtypeStruct((N, D), jnp.float32),
               jax.ShapeDtypeStruct((N, D), jnp.float32)],
)

_tc2 = pl.pallas_call(
    _tc2_body,
    grid=_GRID,
    in_specs=[_part_spec(D), _row_spec((R, D)), _row_spec((R, D)),
              _full_spec((1, D)), _full_spec((1, D)), _full_spec((1, D)),
              _full_spec((D, D))],
    out_specs=_row_spec((R, D)),
    out_shape=jax.ShapeDtypeStruct((N, D), jnp.float32),
)

_tc3 = pl.pallas_call(
    _tc3_body,
    grid=_GRID,
    in_specs=[_part_spec(D), _row_spec((R, D)), _row_spec((R, D)),
              _full_spec((1, D)), _full_spec((1, D)), _full_spec((1, D)),
              _full_spec((D, D // 2)), _full_spec((1, D // 2)),
              _full_spec((D // 2, 40)), _full_spec((1, 40))],
    out_specs=_row_spec((R, 40)),
    out_shape=jax.ShapeDtypeStruct((N, 40), jnp.float32),
)


def kernel(x, edge_index, W1, b1, ln1_w, ln1_b, W2, b2, ln2_w, ln2_b,
           fc1_w, fc1_b, fc2_w, fc2_b):
    src = edge_index[0].astype(jnp.int32)
    dst = edge_index[1].astype(jnp.int32)
    npad = EPAD - E
    src_p = jnp.concatenate(
        [src, jnp.arange(npad, dtype=jnp.int32) % B])
    dst_p = jnp.concatenate(
        [dst, N + (jnp.arange(npad, dtype=jnp.int32) % NDUMP)])
    src_idx = src_p.reshape(NW, STEPS, B)
    dst_idx = dst_p.reshape(NW, STEPS, B)

    _sc_conv = _build_sc_kernels()
    deg_src = jnp.broadcast_to(jnp.arange(B, dtype=jnp.int32),
                               (NW, STEPS, B))
    degp = _sc_conv(jnp.ones((N, D), jnp.float32), deg_src, dst_idx)
    hs1, dinv = _tc1(x, W1, degp)
    acc1 = _sc_conv(hs1, src_idx, dst_idx)
    hs2 = _tc2(acc1, hs1, dinv, b1.reshape(1, D), ln1_w.reshape(1, D),
               ln1_b.reshape(1, D), W2)
    acc2 = _sc_conv(hs2, src_idx, dst_idx)
    out = _tc3(acc2, hs2, dinv, b2.reshape(1, D), ln2_w.reshape(1, D),
               ln2_b.reshape(1, D), fc1_w, fc1_b.reshape(1, D // 2),
               fc2_w, fc2_b.reshape(1, 40))
    return out


# speedup vs baseline: 33.6657x; 1.3096x over previous
import functools

import jax
import jax.numpy as jnp
from jax import lax
from jax.experimental import pallas as pl
from jax.experimental.pallas import tpu as pltpu
from jax.experimental.pallas import tpu_sc as plsc

N = 10000
D = 128
E = 320000
NC = 2
NS = 16
NW = NC * NS
B = 128
STEPS = 80
CHUNKS = 2
CSTEPS = STEPS // CHUNKS
EPAD = NW * STEPS * B
NDUMP = 240
NPAD = N + NDUMP
RPT = NPAD // NS
R = 2000


@functools.lru_cache(maxsize=1)
def _build_sc_kernels():
    mesh = plsc.VectorSubcoreMesh(core_axis_name="c", subcore_axis_name="s",
                                  num_cores=NC, num_subcores=NS)

    @functools.partial(
        pl.kernel,
        out_type=jax.ShapeDtypeStruct((NC, NPAD, D), jnp.float32),
        mesh=mesh,
        scratch_types=[
            pltpu.VMEM((CSTEPS, B), jnp.int32),
            pltpu.VMEM((CSTEPS, B), jnp.int32),
            pltpu.VMEM((B, D), jnp.float32),
            pltpu.VMEM((B, D), jnp.float32),
            pltpu.VMEM_SHARED((NPAD, D), jnp.float32),
            pltpu.SemaphoreType.DMA,
            pltpu.SemaphoreType.DMA,
        ],
    )
    def sc_conv(hs_hbm, src_hbm, dst_hbm, out_hbm,
                src_v, dst_v, buf0, buf1, acc_sh, sem0, sem1):
        c = lax.axis_index("c")
        s = lax.axis_index("s")
        zv = jnp.zeros((16,), jnp.float32)

        def fill_zero(j, carry):
            for g in range(8):
                buf0[j, pl.ds(16 * g, 16)] = zv
            return carry

        lax.fori_loop(0, B, fill_zero, 0)
        for q in range(RPT // B):
            pltpu.sync_copy(buf0, acc_sh.at[pl.ds(s * RPT + q * B, B)])
        bufs = (buf0, buf1)
        sems = (sem0, sem1)
        plsc.subcore_barrier()

        wid = s * NC + c
        for chunk in range(CHUNKS):
            pltpu.sync_copy(src_hbm.at[wid, pl.ds(chunk * CSTEPS, CSTEPS)],
                            src_v)
            pltpu.sync_copy(dst_hbm.at[wid, pl.ds(chunk * CSTEPS, CSTEPS)],
                            dst_v)
            pltpu.async_copy(hs_hbm.at[src_v.at[0]], buf0, sem0)
            pltpu.async_copy(hs_hbm.at[src_v.at[1]], buf1, sem1)

            def step(g, carry):
                for b in range(2):
                    j = g * 2 + b
                    pltpu.make_async_copy(hs_hbm.at[src_v.at[j]], bufs[b],
                                          sems[b]).wait()
                    pltpu.sync_copy(bufs[b], acc_sh.at[dst_v.at[j]], add=True)
                    nxt = j + 2

                    @pl.when(nxt < CSTEPS)
                    def _():
                        pltpu.async_copy(hs_hbm.at[src_v.at[nxt]], bufs[b],
                                         sems[b])
                return carry

            lax.fori_loop(0, CSTEPS // 2, step, 0)
        plsc.subcore_barrier()
        pltpu.sync_copy(acc_sh.at[pl.ds(s * RPT, RPT)],
                        out_hbm.at[c, pl.ds(s * RPT, RPT)])

    return sc_conv


def _tc1_body(x_ref, w1_ref, degp_ref, hs1_ref, dinv_ref):
    deg = degp_ref[0] + degp_ref[1] + 1.0
    dinv = lax.rsqrt(deg)
    h = jnp.dot(x_ref[...], w1_ref[...], preferred_element_type=jnp.float32)
    hs1_ref[...] = h * dinv
    dinv_ref[...] = dinv


def _tc2_body(accp_ref, hs_ref, dinv_ref, b_ref, lnw_ref, lnb_ref, w2_ref,
              out_ref):
    dinv = dinv_ref[...]
    acc = accp_ref[0] + accp_ref[1] + hs_ref[...]
    out = acc * dinv + b_ref[...]
    mu = jnp.mean(out, axis=-1, keepdims=True)
    var = jnp.mean((out - mu) ** 2, axis=-1, keepdims=True)
    h = (out - mu) * lax.rsqrt(var + 1e-5) * lnw_ref[...] + lnb_ref[...]
    h = jnp.maximum(h, 0.0)
    h2 = jnp.dot(h, w2_ref[...], preferred_element_type=jnp.float32)
    out_ref[...] = h2 * dinv


def _tc3_body(accp_ref, hs_ref, dinv_ref, b_ref, lnw_ref, lnb_ref,
              fc1w_ref, fc1b_ref, fc2w_ref, fc2b_ref, out_ref):
    acc = accp_ref[0] + accp_ref[1] + hs_ref[...]
    out = acc * dinv_ref[...] + b_ref[...]
    mu = jnp.mean(out, axis=-1, keepdims=True)
    var = jnp.mean((out - mu) ** 2, axis=-1, keepdims=True)
    h = (out - mu) * lax.rsqrt(var + 1e-5) * lnw_ref[...] + lnb_ref[...]
    h = jnp.maximum(h, 0.0)
    t = jnp.dot(h, fc1w_ref[...], preferred_element_type=jnp.float32)
    t = jnp.maximum(t + fc1b_ref[...], 0.0)
    logits = jnp.dot(t, fc2w_ref[...], preferred_element_type=jnp.float32)
    logits = logits + fc2b_ref[...]
    m = jnp.max(logits, axis=-1, keepdims=True)
    lse = m + jnp.log(jnp.sum(jnp.exp(logits - m), axis=-1, keepdims=True))
    out_ref[...] = logits - lse


def _row_spec(shape):
    return pl.BlockSpec(shape, lambda i: (i, 0))


def _full_spec(shape):
    return pl.BlockSpec(shape, lambda i: tuple(0 for _ in shape))


def _part_spec(last):
    return pl.BlockSpec((NC, R, last), lambda i: (0, i, 0))


_GRID = (N // R,)

_tc1 = pl.pallas_call(
    _tc1_body,
    grid=_GRID,
    in_specs=[_row_spec((R, D)), _full_spec((D, D)), _part_spec(D)],
    out_specs=[_row_spec((R, D)), _row_spec((R, D))],
    out_shape=[jax.ShapeDtypeStruct((N, D), jnp.float32),
               jax.ShapeDtypeStruct((N, D), jnp.float32)],
)

_tc2 = pl.pallas_call(
    _tc2_body,
    grid=_GRID,
    in_specs=[_part_spec(D), _row_spec((R, D)), _row_spec((R, D)),
              _full_spec((1, D)), _full_spec((1, D)), _full_spec((1, D)),
              _full_spec((D, D))],
    out_specs=_row_spec((R, D)),
    out_shape=jax.ShapeDtypeStruct((N, D), jnp.float32),
)

_tc3 = pl.pallas_call(
    _tc3_body,
    grid=_GRID,
    in_specs=[_part_spec(D), _row_spec((R, D)), _row_spec((R, D)),
              _full_spec((1, D)), _full_spec((1, D)), _full_spec((1, D)),
              _full_spec((D, D // 2)), _full_spec((1, D // 2)),
              _full_spec((D // 2, 40)), _full_spec((1, 40))],
    out_specs=_row_spec((R, 40)),
    out_shape=jax.ShapeDtypeStruct((N, 40), jnp.float32),
)


def kernel(x, edge_index, W1, b1, ln1_w, ln1_b, W2, b2, ln2_w, ln2_b,
           fc1_w, fc1_b, fc2_w, fc2_b):
    src = edge_index[0].astype(jnp.int32)
    dst = edge_index[1].astype(jnp.int32)
    npad = EPAD - E
    src_p = jnp.concatenate(
        [src, jnp.arange(npad, dtype=jnp.int32) % B])
    dst_p = jnp.concatenate(
        [dst, N + (jnp.arange(npad, dtype=jnp.int32) % NDUMP)])
    src_idx = src_p.reshape(NW, STEPS, B)
    dst_idx = dst_p.reshape(NW, STEPS, B)

    _sc_conv = _build_sc_kernels()
    degp = _sc_conv(jnp.ones((N, D), jnp.float32), src_idx, dst_idx)
    hs1, dinv = _tc1(x, W1, degp)
    acc1 = _sc_conv(hs1, src_idx, dst_idx)
    hs2 = _tc2(acc1, hs1, dinv, b1.reshape(1, D), ln1_w.reshape(1, D),
               ln1_b.reshape(1, D), W2)
    acc2 = _sc_conv(hs2, src_idx, dst_idx)
    out = _tc3(acc2, hs2, dinv, b2.reshape(1, D), ln2_w.reshape(1, D),
               ln2_b.reshape(1, D), fc1_w, fc1_b.reshape(1, D // 2),
               fc2_w, fc2_b.reshape(1, 40))
    return out
